# baseline (speedup 1.0000x reference)
import functools

import jax
import jax.numpy as jnp
from jax import lax
from jax.experimental import pallas as pl
from jax.experimental.pallas import tpu as pltpu
from jax.experimental.pallas import tpu_sc as plsc

B = 16384
D = 64
H = 128
V = 1000000

_NC = 2
_NS = 16
NW = _NC * _NS
B_PER_W = B // NW
GRP = 16
NGRP = B_PER_W // GRP
WAVE = 4

_TAIL_BASE = (V // 128) * 128
_LAST_SLAB = _TAIL_BASE - 128


@functools.partial(
    pl.kernel,
    mesh=plsc.VectorSubcoreMesh(core_axis_name="c", subcore_axis_name="s"),
    out_type=jax.ShapeDtypeStruct((NW, B_PER_W, D), jnp.float32),
    scratch_types=[
        pltpu.VMEM((B_PER_W + GRP,), jnp.int32),
        pltpu.VMEM((2 * WAVE, D, 128), jnp.float32),
        pltpu.VMEM((D, D), jnp.float32),
        pltpu.VMEM((B_PER_W // 2, D), jnp.float32),
        pltpu.SemaphoreType.DMA((2 * WAVE,)),
        pltpu.SemaphoreType.DMA,
    ],
    compiler_params=pltpu.CompilerParams(needs_layout_passes=False),
)
def _sc_gather(x_hbm, idx_hbm, out_hbm, idx_v, bufs_v, tail_v, out_v, sem, sem2):
    wid = lax.axis_index("s") * _NC + lax.axis_index("c")
    pltpu.sync_copy(idx_hbm.at[wid], idx_v.at[pl.ds(0, B_PER_W)])
    tail_cp = pltpu.async_copy(
        x_hbm.at[:, pl.ds(_TAIL_BASE, V - _TAIL_BASE)], tail_v, sem2)
    tail_cp.wait()

    lanes = lax.iota(jnp.int32, 16)

    def fire(s, u):
        cb = jnp.clip((s // 128) * 128, 0, _LAST_SLAB)
        cb = pl.multiple_of(cb, 128)
        return pltpu.async_copy(
            x_hbm.at[:, pl.ds(cb, 128)], bufs_v.at[u], sem.at[u])

    def select(s, u, j):
        cb = jnp.minimum((s // 128) * 128, _LAST_SLAB)
        col_main = jnp.full((16,), jnp.minimum(s - cb, 127), jnp.int32)
        for q in range(D // 16):
            rows = lanes + (16 * q)
            v_main = plsc.load_gather(bufs_v.at[u], [rows, col_main])
            out_v[j, pl.ds(16 * q, 16)] = v_main

        @pl.when(s >= _TAIL_BASE)
        def _():
            col_tail = jnp.full((16,), s - _TAIL_BASE, jnp.int32)
            for q in range(D // 16):
                rows = lanes + (16 * q)
                v_tail = plsc.load_gather(tail_v, [rows, col_tail])
                out_v[j, pl.ds(16 * q, 16)] = v_tail

    HB = B_PER_W // 2
    NWAVES = GRP // WAVE

    def make_body(hbase):
        def body(g, carry):
            base = g * GRP
            iv = idx_v[pl.ds(hbase + base, GRP)]
            iv_next = idx_v[pl.ds(hbase + base + GRP, GRP)]
            cps = {}
            for w in range(NWAVES):
                if w + 1 < NWAVES:
                    cps[w + 1] = [
                        fire(iv[(w + 1) * WAVE + u], ((w + 1) % 2) * WAVE + u)
                        for u in range(WAVE)
                    ]
                else:
                    for u in range(WAVE):
                        fire(iv_next[u], u)
                h = w % 2
                for u in range(WAVE):
                    if w == 0:
                        pltpu.make_async_copy(
                            x_hbm.at[:, pl.ds(0, 128)], bufs_v.at[u], sem.at[u]
                        ).wait()
                    else:
                        cps[w][u].wait()
                    select(iv[w * WAVE + u], h * WAVE + u, base + w * WAVE + u)
            return carry
        return body

    for half in range(2):
        hbase = half * HB
        iv0 = idx_v[pl.ds(hbase, GRP)]
        for u in range(WAVE):
            fire(iv0[u], u)
        lax.fori_loop(0, HB // GRP, make_body(hbase), 0)
        for u in range(WAVE):
            pltpu.make_async_copy(
                x_hbm.at[:, pl.ds(0, 128)], bufs_v.at[u], sem.at[u]).wait()
        pltpu.sync_copy(out_v, out_hbm.at[wid, pl.ds(hbase, HB)])


def _tc_body(x_ref, wt_ref, b_ref, o_ref):
    acc = jnp.dot(x_ref[...], wt_ref[...], preferred_element_type=jnp.float32)
    acc = acc + b_ref[...]
    o_ref[...] = acc / (1.0 + jnp.abs(acc))


_BM = 2048


def _tc_linear_softsign(x, wt, b2d):
    return pl.pallas_call(
        _tc_body,
        grid=(B // _BM,),
        in_specs=[
            pl.BlockSpec((_BM, D), lambda i: (i, 0)),
            pl.BlockSpec((D, H), lambda i: (0, 0)),
            pl.BlockSpec((1, H), lambda i: (0, 0)),
        ],
        out_specs=pl.BlockSpec((_BM, H), lambda i: (i, 0)),
        out_shape=jax.ShapeDtypeStruct((B, H), jnp.float32),
    )(x, wt, b2d)


def kernel(spk_id, embedding_table, W, b):
    x = embedding_table.T
    sid = spk_id.astype(jnp.int32)
    idx = sid.reshape(NW, B_PER_W)
    rows = _sc_gather(x, idx)
    return _tc_linear_softsign(rows.reshape(B, D), W.T, b.reshape(1, H))


# speedup vs baseline: 1.1613x; 1.1613x over previous
import functools

import jax
import jax.numpy as jnp
from jax import lax
from jax.experimental import pallas as pl
from jax.experimental.pallas import tpu as pltpu
from jax.experimental.pallas import tpu_sc as plsc

B = 16384
D = 64
H = 128
V = 1000000

_NC = 2
_NS = 16
NW = _NC * _NS
B_PER_W = B // NW
GRP = 16
NGRP = B_PER_W // GRP
WAVE = 4

_TAIL_BASE = (V // 128) * 128
_LAST_SLAB = _TAIL_BASE - 128


@functools.partial(
    pl.kernel,
    mesh=plsc.VectorSubcoreMesh(core_axis_name="c", subcore_axis_name="s"),
    out_type=jax.ShapeDtypeStruct((NW, B_PER_W, D), jnp.float32),
    scratch_types=[
        pltpu.VMEM((B_PER_W + GRP,), jnp.int32),
        pltpu.VMEM((2 * WAVE, D, 128), jnp.float32),
        pltpu.VMEM((D, D), jnp.float32),
        pltpu.VMEM((B_PER_W // 2, D), jnp.float32),
        pltpu.SemaphoreType.DMA,
        pltpu.SemaphoreType.DMA,
    ],
    compiler_params=pltpu.CompilerParams(needs_layout_passes=False),
)
def _sc_gather(x_hbm, idx_hbm, out_hbm, idx_v, bufs_v, tail_v, out_v, sem, sem2):
    wid = lax.axis_index("s") * _NC + lax.axis_index("c")
    pltpu.sync_copy(idx_hbm.at[wid], idx_v.at[pl.ds(0, B_PER_W)])
    tail_cp = pltpu.async_copy(
        x_hbm.at[:, pl.ds(_TAIL_BASE, V - _TAIL_BASE)], tail_v, sem2)
    tail_cp.wait()

    lanes = lax.iota(jnp.int32, 16)

    def fire(s, u):
        cb = jnp.clip((s // 128) * 128, 0, _LAST_SLAB)
        cb = pl.multiple_of(cb, 128)
        return pltpu.async_copy(
            x_hbm.at[:, pl.ds(cb, 128)], bufs_v.at[u], sem)

    def select(s, u, j):
        cb = jnp.minimum((s // 128) * 128, _LAST_SLAB)
        col_main = jnp.full((16,), jnp.minimum(s - cb, 127), jnp.int32)
        for q in range(D // 16):
            rows = lanes + (16 * q)
            v_main = plsc.load_gather(bufs_v.at[u], [rows, col_main])
            out_v[j, pl.ds(16 * q, 16)] = v_main

        @pl.when(s >= _TAIL_BASE)
        def _():
            col_tail = jnp.full((16,), s - _TAIL_BASE, jnp.int32)
            for q in range(D // 16):
                rows = lanes + (16 * q)
                v_tail = plsc.load_gather(tail_v, [rows, col_tail])
                out_v[j, pl.ds(16 * q, 16)] = v_tail

    HB = B_PER_W // 2
    NWAVES = GRP // WAVE

    def make_body(hbase):
        def body(g, carry):
            base = g * GRP
            iv = idx_v[pl.ds(hbase + base, GRP)]
            iv_next = idx_v[pl.ds(hbase + base + GRP, GRP)]
            cps = {}
            for w in range(NWAVES):
                if w + 1 < NWAVES:
                    cps[w + 1] = [
                        fire(iv[(w + 1) * WAVE + u], ((w + 1) % 2) * WAVE + u)
                        for u in range(WAVE)
                    ]
                else:
                    for u in range(WAVE):
                        fire(iv_next[u], u)
                h = w % 2
                for u in range(WAVE):
                    if w == 0:
                        pltpu.make_async_copy(
                            x_hbm.at[:, pl.ds(0, 128)], bufs_v.at[u], sem
                        ).wait()
                    else:
                        cps[w][u].wait()
                    select(iv[w * WAVE + u], h * WAVE + u, base + w * WAVE + u)
            return carry
        return body

    for half in range(2):
        hbase = half * HB
        iv0 = idx_v[pl.ds(hbase, GRP)]
        for u in range(WAVE):
            fire(iv0[u], u)
        lax.fori_loop(0, HB // GRP, make_body(hbase), 0)
        for u in range(WAVE):
            pltpu.make_async_copy(
                x_hbm.at[:, pl.ds(0, 128)], bufs_v.at[u], sem).wait()
        pltpu.sync_copy(out_v, out_hbm.at[wid, pl.ds(hbase, HB)])


def _tc_body(x_ref, wt_ref, b_ref, o_ref):
    acc = jnp.dot(x_ref[...], wt_ref[...], preferred_element_type=jnp.float32)
    acc = acc + b_ref[...]
    o_ref[...] = acc / (1.0 + jnp.abs(acc))


_BM = 2048


def _tc_linear_softsign(x, wt, b2d):
    return pl.pallas_call(
        _tc_body,
        grid=(B // _BM,),
        in_specs=[
            pl.BlockSpec((_BM, D), lambda i: (i, 0)),
            pl.BlockSpec((D, H), lambda i: (0, 0)),
            pl.BlockSpec((1, H), lambda i: (0, 0)),
        ],
        out_specs=pl.BlockSpec((_BM, H), lambda i: (i, 0)),
        out_shape=jax.ShapeDtypeStruct((B, H), jnp.float32),
    )(x, wt, b2d)


def kernel(spk_id, embedding_table, W, b):
    x = embedding_table.T
    sid = spk_id.astype(jnp.int32)
    idx = sid.reshape(NW, B_PER_W)
    rows = _sc_gather(x, idx)
    return _tc_linear_softsign(rows.reshape(B, D), W.T, b.reshape(1, H))
